# Initial kernel scaffold; baseline (speedup 1.0000x reference)
import jax
import jax.numpy as jnp
from jax.experimental import pallas as pl


def kernel(x, edge_attr, W1, root1, bias1, g1, b1, W2, root2, bias2, g2, b2, Ws, roots, biass, gs, bs, edge_index):
    raise NotImplementedError("write your pallas kernel here")


# speedup vs baseline: 3.1913x; 3.1913x over previous
import functools

import jax
import jax.numpy as jnp
from jax import lax
from jax.experimental import pallas as pl
from jax.experimental.pallas import tpu as pltpu
from jax.experimental.pallas import tpu_sc as plsc

F32 = jnp.float32
I32 = jnp.int32

NC = 2
NS = 16
NW = NC * NS
LANES = 16

CH = 80


def _prep_body(ei_ref, at_ref, gidx_ref, wts_ref):
    src = ei_ref[0:1, :]
    v0 = at_ref[0:1, :] * 4.0
    v1 = at_ref[1:2, :] * 4.0
    lo0 = jnp.floor(v0)
    lo1 = jnp.floor(v1)
    f0 = v0 - lo0
    f1 = v1 - lo1
    i0 = jnp.clip(lo0.astype(I32), 0, 4)
    j0 = jnp.clip(i0 + 1, 0, 4)
    i1 = jnp.clip(lo1.astype(I32), 0, 4)
    j1 = jnp.clip(i1 + 1, 0, 4)
    base = src * 25
    gidx_ref[0:1, :] = base + i0 * 5 + i1
    gidx_ref[1:2, :] = base + i0 * 5 + j1
    gidx_ref[2:3, :] = base + j0 * 5 + i1
    gidx_ref[3:4, :] = base + j0 * 5 + j1
    w0a = 1.0 - f0
    w1a = 1.0 - f1
    wts_ref[0:1, :] = w0a * w1a
    wts_ref[1:2, :] = w0a * f1
    wts_ref[2:3, :] = f0 * w1a
    wts_ref[3:4, :] = f0 * f1


def _mm1_body(x_ref, w_ref, r_ref, xw_ref, xr_ref):
    x = x_ref[...]
    xw_ref[...] = jnp.dot(x, w_ref[...], preferred_element_type=F32)
    xr_ref[...] = jnp.dot(x, r_ref[...], preferred_element_type=F32)


def _stats1_body(a_ref, d_ref, xr_ref, b_ref, o_ref, ps_ref, pq_ref):
    acc = a_ref[0] + a_ref[1]
    deg1 = jnp.maximum(d_ref[0][:, 0:1] + d_ref[1][:, 0:1], 1.0)
    o = acc / deg1 + xr_ref[...] + b_ref[...]
    o_ref[...] = o
    ps_ref[...] = jnp.broadcast_to(jnp.sum(o, axis=0, keepdims=True), (8, 128))[None]
    pq_ref[...] = jnp.broadcast_to(jnp.sum(o * o, axis=0, keepdims=True), (8, 128))[None]


def _mm2_body(o_ref, sc_ref, sh_ref, w_ref, r_ref, xw_ref, hr_ref):
    o = o_ref[...] * sc_ref[...] + sh_ref[...]
    h = jnp.where(o > 0.0, o, jnp.exp(jnp.minimum(o, 0.0)) - 1.0)
    xw_ref[...] = jnp.dot(h, w_ref[...], preferred_element_type=F32)
    hr_ref[...] = jnp.dot(h, r_ref[...], preferred_element_type=F32)


def _stats2_body(a_ref, d_ref, hr_ref, b2_ref, as_ref, x_ref, ws_ref,
                 rs_ref, bs_ref, o2_ref, os_ref, ps2_ref, pq2_ref,
                 pss_ref, pqs_ref):
    deg1 = jnp.maximum(d_ref[0][:, 0:1] + d_ref[1][:, 0:1], 1.0)
    o2 = (a_ref[0] + a_ref[1]) / deg1 + hr_ref[...] + b2_ref[...]
    o2_ref[...] = o2
    asum = (as_ref[0] + as_ref[1]) / deg1
    os_ = (jnp.dot(asum, ws_ref[...], preferred_element_type=F32)
           + jnp.dot(x_ref[...], rs_ref[...], preferred_element_type=F32)
           + bs_ref[...])
    os_ref[...] = os_
    ps2_ref[...] = jnp.broadcast_to(jnp.sum(o2, axis=0, keepdims=True), (8, 128))[None]
    pq2_ref[...] = jnp.broadcast_to(jnp.sum(o2 * o2, axis=0, keepdims=True), (8, 128))[None]
    pss_ref[...] = jnp.broadcast_to(jnp.sum(os_, axis=0, keepdims=True), (8, 128))[None]
    pqs_ref[...] = jnp.broadcast_to(jnp.sum(os_ * os_, axis=0, keepdims=True), (8, 128))[None]


def _final_body(o2_ref, os_ref, sc2_ref, sh2_ref, scs_ref, shs_ref, out_ref):
    h = o2_ref[...] * sc2_ref[...] + sh2_ref[...]
    s = os_ref[...] * scs_ref[...] + shs_ref[...]
    t = h + s
    out_ref[...] = jnp.where(t > 0.0, t, jnp.exp(jnp.minimum(t, 0.0)) - 1.0)


def _copy_out(src_sp, dst_hbm, cid, sid, n_nodes):
    nblk16 = n_nodes // 16
    per = nblk16 // NS
    nblk = jnp.where(sid == NS - 1, nblk16 - per * (NS - 1), per)

    def body(j, _):
        r = (sid * per + j) * 16
        pltpu.sync_copy(src_sp.at[pl.ds(r, 16)], dst_hbm.at[cid, pl.ds(r, 16)])
        return 0
    lax.fori_loop(0, nblk, body, 0)


def _zero_acc(zrow_v, acc_sp, sid, n_nodes, width):
    def zbody(r, _):
        for v in range(width // LANES):
            zrow_v[r, pl.ds(v * LANES, LANES)] = jnp.zeros((LANES,), F32)
        return 0
    lax.fori_loop(0, 16, zbody, 0)
    nblk16 = n_nodes // 16
    per = nblk16 // NS
    nblk = jnp.where(sid == NS - 1, nblk16 - per * (NS - 1), per)

    def body(j, _):
        r = (sid * per + j) * 16
        pltpu.sync_copy(zrow_v, acc_sp.at[pl.ds(r, 16)])
        return 0
    lax.fori_loop(0, nblk, body, 0)


def _sc_edge_sum_body(n_nodes, ew, e_total, x_hbm, ei_hbm, outs_hbm,
                      src_v, dst_v, rows_v, zrow_v, accs, sem):
    cid = lax.axis_index("c")
    sid = lax.axis_index("s")
    wid = cid * NS + sid
    _zero_acc(zrow_v, accs, sid, n_nodes, 128)
    plsc.subcore_barrier()

    ebase = wid * ew

    def chunk(i, _):
        b = ebase + i * CH
        pltpu.sync_copy(ei_hbm.at[pl.ds(b, CH)], src_v)
        pltpu.sync_copy(ei_hbm.at[pl.ds(e_total + b, CH)], dst_v)
        pltpu.async_copy(x_hbm.at[src_v], rows_v, sem).wait()
        pltpu.sync_copy(rows_v, accs.at[dst_v], add=True)
        return 0
    lax.fori_loop(0, ew // CH, chunk, 0)
    plsc.subcore_barrier()
    _copy_out(accs, outs_hbm, cid, sid, n_nodes)


def _sc_deg_body(n_nodes, ew, e_total, ei_hbm, outd_hbm,
                 dst_v, ones_v, zrow_v, accd):
    cid = lax.axis_index("c")
    sid = lax.axis_index("s")
    wid = cid * NS + sid
    _zero_acc(zrow_v, accd, sid, n_nodes, 128)

    def ones_body(r, _):
        for v in range(8):
            ones_v[r, pl.ds(v * LANES, LANES)] = jnp.ones((LANES,), F32)
        return 0
    lax.fori_loop(0, CH, ones_body, 0)
    plsc.subcore_barrier()

    ebase = wid * ew

    def chunk(i, _):
        b = ebase + i * CH
        pltpu.sync_copy(ei_hbm.at[pl.ds(e_total + b, CH)], dst_v)
        pltpu.sync_copy(ones_v, accd.at[dst_v], add=True)
        return 0
    lax.fori_loop(0, ew // CH, chunk, 0)
    plsc.subcore_barrier()
    _copy_out(accd, outd_hbm, cid, sid, n_nodes)


def _sc_conv_body(n_nodes, ew, e_total, tw_hbm, gidx_hbm, wts_hbm, ei_hbm, out_hbm,
                  i0_v, i1_v, i2_v, i3_v, w0_v, w1_v, w2_v, w3_v, dst_v,
                  g0, g1, g2, g3, zrow_v, acc, s0, s1, s2, s3):
    cid = lax.axis_index("c")
    sid = lax.axis_index("s")
    wid = cid * NS + sid
    _zero_acc(zrow_v, acc, sid, n_nodes, 128)
    plsc.subcore_barrier()

    ebase = wid * ew

    def chunk(i, _):
        b = ebase + i * CH
        pltpu.sync_copy(gidx_hbm.at[pl.ds(b, CH)], i0_v)
        pltpu.sync_copy(gidx_hbm.at[pl.ds(e_total + b, CH)], i1_v)
        pltpu.sync_copy(gidx_hbm.at[pl.ds(2 * e_total + b, CH)], i2_v)
        pltpu.sync_copy(gidx_hbm.at[pl.ds(3 * e_total + b, CH)], i3_v)
        pltpu.sync_copy(wts_hbm.at[pl.ds(b, CH)], w0_v)
        pltpu.sync_copy(wts_hbm.at[pl.ds(e_total + b, CH)], w1_v)
        pltpu.sync_copy(wts_hbm.at[pl.ds(2 * e_total + b, CH)], w2_v)
        pltpu.sync_copy(wts_hbm.at[pl.ds(3 * e_total + b, CH)], w3_v)
        pltpu.sync_copy(ei_hbm.at[pl.ds(e_total + b, CH)], dst_v)
        d0 = pltpu.async_copy(tw_hbm.at[i0_v], g0, s0)
        d1 = pltpu.async_copy(tw_hbm.at[i1_v], g1, s1)
        d2 = pltpu.async_copy(tw_hbm.at[i2_v], g2, s2)
        d3 = pltpu.async_copy(tw_hbm.at[i3_v], g3, s3)
        d0.wait()
        d1.wait()
        d2.wait()
        d3.wait()

        def group(g, _):
            gb = g * LANES
            wv0 = w0_v[pl.ds(gb, LANES)]
            wv1 = w1_v[pl.ds(gb, LANES)]
            wv2 = w2_v[pl.ds(gb, LANES)]
            wv3 = w3_v[pl.ds(gb, LANES)]

            def lane(l, _):
                e = gb + l
                lv = jnp.full((LANES,), l, I32)
                wb0 = wv0.at[lv].get(mode="promise_in_bounds")
                wb1 = wv1.at[lv].get(mode="promise_in_bounds")
                wb2 = wv2.at[lv].get(mode="promise_in_bounds")
                wb3 = wv3.at[lv].get(mode="promise_in_bounds")
                for v in range(8):
                    sl = pl.ds(v * LANES, LANES)
                    g0[e, sl] = (g0[e, sl] * wb0 + g1[e, sl] * wb1
                                 + g2[e, sl] * wb2 + g3[e, sl] * wb3)
                return 0
            lax.fori_loop(0, LANES, lane, 0)
            return 0
        lax.fori_loop(0, CH // LANES, group, 0)
        pltpu.sync_copy(g0, acc.at[dst_v], add=True)
        return 0
    lax.fori_loop(0, ew // CH, chunk, 0)
    plsc.subcore_barrier()
    _copy_out(acc, out_hbm, cid, sid, n_nodes)


def kernel(x, edge_attr, W1, root1, bias1, g1, b1, W2, root2, bias2, g2, b2,
           Ws, roots, biass, gs, bs, edge_index):
    n, c = x.shape
    e = edge_index.shape[1]
    kd = W1.shape[0]
    ew = e // NW
    bn = 400
    gn = n // bn
    be = 16000
    ge = e // be

    mesh = plsc.VectorSubcoreMesh(core_axis_name="c", subcore_axis_name="s",
                                  num_cores=NC, num_subcores=NS)

    prep = pl.pallas_call(
        _prep_body,
        grid=(ge,),
        in_specs=[
            pl.BlockSpec((2, be), lambda i: (0, i)),
            pl.BlockSpec((2, be), lambda i: (0, i)),
        ],
        out_specs=[
            pl.BlockSpec((4, be), lambda i: (0, i)),
            pl.BlockSpec((4, be), lambda i: (0, i)),
        ],
        out_shape=[
            jax.ShapeDtypeStruct((4, e), I32),
            jax.ShapeDtypeStruct((4, e), F32),
        ],
    )
    gidx, wts = prep(edge_index, edge_attr.T)

    w1f = W1.transpose(1, 0, 2).reshape(c, kd * c)
    w2f = W2.transpose(1, 0, 2).reshape(c, kd * c)
    mm1 = pl.pallas_call(
        _mm1_body,
        grid=(gn,),
        in_specs=[
            pl.BlockSpec((bn, c), lambda i: (i, 0)),
            pl.BlockSpec((c, kd * c), lambda i: (0, 0)),
            pl.BlockSpec((c, c), lambda i: (0, 0)),
        ],
        out_specs=[
            pl.BlockSpec((bn, kd * c), lambda i: (i, 0)),
            pl.BlockSpec((bn, c), lambda i: (i, 0)),
        ],
        out_shape=[
            jax.ShapeDtypeStruct((n, kd * c), F32),
            jax.ShapeDtypeStruct((n, c), F32),
        ],
    )
    xw1, xr1 = mm1(x, w1f, root1)

    edge_sum = functools.partial(
        pl.kernel,
        out_type=jax.ShapeDtypeStruct((NC, n, c), F32),
        mesh=mesh,
        scratch_types=[
            pltpu.VMEM((CH,), I32),
            pltpu.VMEM((CH,), I32),
            pltpu.VMEM((CH, c), F32),
            pltpu.VMEM((16, c), F32),
            pltpu.VMEM_SHARED((n, c), F32),
            pltpu.SemaphoreType.DMA,
        ],
    )(functools.partial(_sc_edge_sum_body, n, ew, e))
    acc_s = edge_sum(x, edge_index.reshape(-1))

    deg_kernel = functools.partial(
        pl.kernel,
        out_type=jax.ShapeDtypeStruct((NC, n, c), F32),
        mesh=mesh,
        scratch_types=[
            pltpu.VMEM((CH,), I32),
            pltpu.VMEM((CH, c), F32),
            pltpu.VMEM((16, c), F32),
            pltpu.VMEM_SHARED((n, c), F32),
        ],
    )(functools.partial(_sc_deg_body, n, ew, e))
    deg_t = deg_kernel(edge_index.reshape(-1))

    def conv_pass(table):
        f = functools.partial(
            pl.kernel,
            out_type=jax.ShapeDtypeStruct((NC, n, c), F32),
            mesh=mesh,
            scratch_types=(
                [pltpu.VMEM((CH,), I32)] * 4
                + [pltpu.VMEM((CH,), F32)] * 4
                + [pltpu.VMEM((CH,), I32)]
                + [pltpu.VMEM((CH, c), F32)] * 4
                + [pltpu.VMEM((16, c), F32)]
                + [pltpu.VMEM_SHARED((n, c), F32)]
                + [pltpu.SemaphoreType.DMA] * 4
            ),
        )(functools.partial(_sc_conv_body, n, ew, e))
        return f(table, gidx.reshape(-1), wts.reshape(-1), edge_index.reshape(-1))

    acc1 = conv_pass(xw1.reshape(n * kd, c))

    stats1 = pl.pallas_call(
        _stats1_body,
        grid=(gn,),
        in_specs=[
            pl.BlockSpec((NC, bn, c), lambda i: (0, i, 0)),
            pl.BlockSpec((NC, bn, c), lambda i: (0, i, 0)),
            pl.BlockSpec((bn, c), lambda i: (i, 0)),
            pl.BlockSpec((1, c), lambda i: (0, 0)),
        ],
        out_specs=[
            pl.BlockSpec((bn, c), lambda i: (i, 0)),
            pl.BlockSpec((1, 8, c), lambda i: (i, 0, 0)),
            pl.BlockSpec((1, 8, c), lambda i: (i, 0, 0)),
        ],
        out_shape=[
            jax.ShapeDtypeStruct((n, c), F32),
            jax.ShapeDtypeStruct((gn, 8, c), F32),
            jax.ShapeDtypeStruct((gn, 8, c), F32),
        ],
    )
    o1, ps1, pq1 = stats1(acc1, deg_t, xr1, bias1.reshape(1, c))

    mu1 = jnp.sum(ps1[:, 0, :], axis=0) / n
    var1 = jnp.sum(pq1[:, 0, :], axis=0) / n - mu1 * mu1
    sc1 = g1 / jnp.sqrt(var1 + 1e-5)
    sh1 = b1 - mu1 * sc1

    mm2 = pl.pallas_call(
        _mm2_body,
        grid=(gn,),
        in_specs=[
            pl.BlockSpec((bn, c), lambda i: (i, 0)),
            pl.BlockSpec((1, c), lambda i: (0, 0)),
            pl.BlockSpec((1, c), lambda i: (0, 0)),
            pl.BlockSpec((c, kd * c), lambda i: (0, 0)),
            pl.BlockSpec((c, c), lambda i: (0, 0)),
        ],
        out_specs=[
            pl.BlockSpec((bn, kd * c), lambda i: (i, 0)),
            pl.BlockSpec((bn, c), lambda i: (i, 0)),
        ],
        out_shape=[
            jax.ShapeDtypeStruct((n, kd * c), F32),
            jax.ShapeDtypeStruct((n, c), F32),
        ],
    )
    xw2, hr2 = mm2(o1, sc1.reshape(1, c), sh1.reshape(1, c), w2f, root2)

    acc2 = conv_pass(xw2.reshape(n * kd, c))

    stats2 = pl.pallas_call(
        _stats2_body,
        grid=(gn,),
        in_specs=[
            pl.BlockSpec((NC, bn, c), lambda i: (0, i, 0)),
            pl.BlockSpec((NC, bn, c), lambda i: (0, i, 0)),
            pl.BlockSpec((bn, c), lambda i: (i, 0)),
            pl.BlockSpec((1, c), lambda i: (0, 0)),
            pl.BlockSpec((NC, bn, c), lambda i: (0, i, 0)),
            pl.BlockSpec((bn, c), lambda i: (i, 0)),
            pl.BlockSpec((c, c), lambda i: (0, 0)),
            pl.BlockSpec((c, c), lambda i: (0, 0)),
            pl.BlockSpec((1, c), lambda i: (0, 0)),
        ],
        out_specs=[
            pl.BlockSpec((bn, c), lambda i: (i, 0)),
            pl.BlockSpec((bn, c), lambda i: (i, 0)),
            pl.BlockSpec((1, 8, c), lambda i: (i, 0, 0)),
            pl.BlockSpec((1, 8, c), lambda i: (i, 0, 0)),
            pl.BlockSpec((1, 8, c), lambda i: (i, 0, 0)),
            pl.BlockSpec((1, 8, c), lambda i: (i, 0, 0)),
        ],
        out_shape=[
            jax.ShapeDtypeStruct((n, c), F32),
            jax.ShapeDtypeStruct((n, c), F32),
            jax.ShapeDtypeStruct((gn, 8, c), F32),
            jax.ShapeDtypeStruct((gn, 8, c), F32),
            jax.ShapeDtypeStruct((gn, 8, c), F32),
            jax.ShapeDtypeStruct((gn, 8, c), F32),
        ],
    )
    o2, os_, ps2, pq2, pss, pqs = stats2(
        acc2, deg_t, hr2, bias2.reshape(1, c), acc_s, x, Ws[0], roots,
        biass.reshape(1, c))

    mu2 = jnp.sum(ps2[:, 0, :], axis=0) / n
    var2 = jnp.sum(pq2[:, 0, :], axis=0) / n - mu2 * mu2
    sc2 = g2 / jnp.sqrt(var2 + 1e-5)
    sh2 = b2 - mu2 * sc2
    mus = jnp.sum(pss[:, 0, :], axis=0) / n
    vars_ = jnp.sum(pqs[:, 0, :], axis=0) / n - mus * mus
    scs = gs / jnp.sqrt(vars_ + 1e-5)
    shs = bs - mus * scs

    final = pl.pallas_call(
        _final_body,
        grid=(gn,),
        in_specs=[
            pl.BlockSpec((bn, c), lambda i: (i, 0)),
            pl.BlockSpec((bn, c), lambda i: (i, 0)),
            pl.BlockSpec((1, c), lambda i: (0, 0)),
            pl.BlockSpec((1, c), lambda i: (0, 0)),
            pl.BlockSpec((1, c), lambda i: (0, 0)),
            pl.BlockSpec((1, c), lambda i: (0, 0)),
        ],
        out_specs=pl.BlockSpec((bn, c), lambda i: (i, 0)),
        out_shape=jax.ShapeDtypeStruct((n, c), F32),
    )
    return final(o2, os_, sc2.reshape(1, c), sh2.reshape(1, c),
                 scs.reshape(1, c), shs.reshape(1, c))
